# baseline (speedup 1.0000x reference)
import functools

import jax
import jax.numpy as jnp
from jax import lax
from jax.experimental import pallas as pl
from jax.experimental.pallas import tpu as pltpu
from jax.experimental.pallas import tpu_sc as plsc

NUM_CORES = 2
NUM_SUBCORES = 16
NUM_WORKERS = NUM_CORES * NUM_SUBCORES
NBUF = 5


@functools.cache
def _make_gather(NB: int, H: int, D: int, V: int):
    assert NB % NUM_WORKERS == 0
    b_per_w = NB // NUM_WORKERS
    assert H % NBUF == 0
    mesh = plsc.VectorSubcoreMesh(core_axis_name="c", subcore_axis_name="s")

    @functools.partial(
        pl.kernel,
        mesh=mesh,
        out_type=jax.ShapeDtypeStruct((H, NB, D), jnp.float32),
        scratch_types=[
            pltpu.VMEM((b_per_w, H), jnp.int32),
            pltpu.VMEM((H, b_per_w), jnp.int32),
            pltpu.VMEM((NBUF, b_per_w, D), jnp.float32),
            pltpu.VMEM_SHARED((V, D), jnp.float32),
            pltpu.SemaphoreType.DMA((NBUF,)),
            pltpu.SemaphoreType.DMA((NBUF,)),
        ],
        compiler_params=pltpu.CompilerParams(needs_layout_passes=False),
    )
    def gather_kernel(table_hbm, words_hbm, out_hbm, idx_slab, idx_v, rows_v,
                      table_sp, gsem, wsem):
        wid = lax.axis_index("s") * NUM_CORES + lax.axis_index("c")
        b0 = pl.multiple_of(wid * b_per_w, b_per_w)

        @pl.when(lax.axis_index("s") == 0)
        def _stage_table():
            pltpu.sync_copy(table_hbm, table_sp)

        pltpu.sync_copy(words_hbm.at[pl.ds(b0, b_per_w)], idx_slab)

        lanes = lax.iota(jnp.int32, 16)

        @pl.loop(0, H)
        def transpose_h(h):
            cols = jnp.broadcast_to(h, (16,)).astype(jnp.int32)
            for j in range(b_per_w // 16):
                vals = plsc.load_gather(idx_slab, [j * 16 + lanes, cols])
                idx_v[h, pl.ds(j * 16, 16)] = vals

        plsc.subcore_barrier()

        def start_gather(h, buf):
            pltpu.async_copy(table_sp.at[idx_v.at[h]], rows_v.at[buf],
                             gsem.at[buf])

        def wait_gather(buf):
            pltpu.make_async_copy(table_sp.at[idx_v.at[0]], rows_v.at[buf],
                                  gsem.at[buf]).wait()

        def start_write(h, buf):
            pltpu.async_copy(rows_v.at[buf],
                             out_hbm.at[h, pl.ds(b0, b_per_w)], wsem.at[buf])

        def wait_write(buf):
            pltpu.make_async_copy(rows_v.at[buf],
                                  out_hbm.at[0, pl.ds(b0, b_per_w)],
                                  wsem.at[buf]).wait()

        for i in range(NBUF - 1):
            start_gather(i, i)
        for i in range(NBUF):
            if i >= 1:
                wait_write((i - 1) % NBUF)
            start_gather(i + NBUF - 1, (i - 1) % NBUF)
            wait_gather(i)
            start_write(i, i)

        @pl.loop(1, H // NBUF - 1)
        def mid(s):
            for i in range(NBUF):
                h = s * NBUF + i
                wait_write((i - 1) % NBUF)
                start_gather(h + NBUF - 1, (i - 1) % NBUF)
                wait_gather(i)
                start_write(h, i)

        h_last = H - NBUF
        for i in range(NBUF):
            if i == 0:
                wait_write((h_last - 1) % NBUF)
                start_gather(H - 1, (H - 1) % NBUF)
            wait_gather(i)
            start_write(h_last + i, i)
        for i in range(NBUF):
            wait_write(i)

    return gather_kernel


def kernel(words, table):
    NB, H = words.shape
    V, D = table.shape
    out = _make_gather(NB, H, D, V)(table, words)
    return out.transpose(1, 0, 2)


# speedup vs baseline: 1.0906x; 1.0906x over previous
import functools

import jax
import jax.numpy as jnp
from jax import lax
from jax.experimental import pallas as pl
from jax.experimental.pallas import tpu as pltpu
from jax.experimental.pallas import tpu_sc as plsc

NUM_CORES = 2
NUM_SUBCORES = 16
NUM_WORKERS = NUM_CORES * NUM_SUBCORES
NBUF = 5


@functools.cache
def _make_gather(NB: int, H: int, D: int, V: int):
    assert NB % NUM_WORKERS == 0
    b_per_w = NB // NUM_WORKERS
    assert H % NBUF == 0
    mesh = plsc.VectorSubcoreMesh(core_axis_name="c", subcore_axis_name="s")

    @functools.partial(
        pl.kernel,
        mesh=mesh,
        out_type=jax.ShapeDtypeStruct((H, NB, D), jnp.float32),
        scratch_types=[
            pltpu.VMEM((H, b_per_w), jnp.int32),
            pltpu.VMEM((NBUF, b_per_w, D), jnp.float32),
            pltpu.VMEM_SHARED((V, D), jnp.float32),
            pltpu.SemaphoreType.DMA((NBUF,)),
            pltpu.SemaphoreType.DMA((NBUF,)),
        ],
    )
    def gather_kernel(table_hbm, idx_hbm, out_hbm, idx_v, rows_v,
                      table_sp, gsem, wsem):
        wid = lax.axis_index("s") * NUM_CORES + lax.axis_index("c")
        b0 = pl.multiple_of(wid * b_per_w, b_per_w)

        @pl.when(lax.axis_index("s") == 0)
        def _stage_table():
            pltpu.sync_copy(table_hbm, table_sp)

        pltpu.sync_copy(idx_hbm.at[wid], idx_v)
        plsc.subcore_barrier()

        def start_gather(h, buf):
            pltpu.async_copy(table_sp.at[idx_v.at[h]], rows_v.at[buf],
                             gsem.at[buf])

        def wait_gather(buf):
            pltpu.make_async_copy(table_sp.at[idx_v.at[0]], rows_v.at[buf],
                                  gsem.at[buf]).wait()

        def start_write(h, buf):
            pltpu.async_copy(rows_v.at[buf],
                             out_hbm.at[h, pl.ds(b0, b_per_w)], wsem.at[buf])

        def wait_write(buf):
            pltpu.make_async_copy(rows_v.at[buf],
                                  out_hbm.at[0, pl.ds(b0, b_per_w)],
                                  wsem.at[buf]).wait()

        for i in range(NBUF - 1):
            start_gather(i, i)
        for i in range(NBUF):
            if i >= 1:
                wait_write((i - 1) % NBUF)
            start_gather(i + NBUF - 1, (i - 1) % NBUF)
            wait_gather(i)
            start_write(i, i)

        @pl.loop(1, H // NBUF - 1)
        def mid(s):
            for i in range(NBUF):
                h = s * NBUF + i
                wait_write((i - 1) % NBUF)
                start_gather(h + NBUF - 1, (i - 1) % NBUF)
                wait_gather(i)
                start_write(h, i)

        h_last = H - NBUF
        for i in range(NBUF):
            if i == 0:
                wait_write((h_last - 1) % NBUF)
                start_gather(H - 1, (H - 1) % NBUF)
            wait_gather(i)
            start_write(h_last + i, i)
        for i in range(NBUF):
            wait_write(i)

    return gather_kernel


def kernel(words, table):
    NB, H = words.shape
    V, D = table.shape
    b_per_w = NB // NUM_WORKERS
    idx = words.reshape(NUM_WORKERS, b_per_w, H).transpose(0, 2, 1)
    out = _make_gather(NB, H, D, V)(table, idx)
    return out.transpose(1, 0, 2)


# speedup vs baseline: 1.0934x; 1.0026x over previous
import functools

import jax
import jax.numpy as jnp
from jax import lax
from jax.experimental import pallas as pl
from jax.experimental.pallas import tpu as pltpu
from jax.experimental.pallas import tpu_sc as plsc

NUM_CORES = 2
NUM_SUBCORES = 16
NUM_WORKERS = NUM_CORES * NUM_SUBCORES
NBUF = 5


@functools.cache
def _make_gather(NB: int, H: int, D: int, V: int):
    assert NB % NUM_WORKERS == 0
    b_per_w = NB // NUM_WORKERS
    assert H % NBUF == 0
    mesh = plsc.VectorSubcoreMesh(core_axis_name="c", subcore_axis_name="s")

    @functools.partial(
        pl.kernel,
        mesh=mesh,
        out_type=jax.ShapeDtypeStruct((H, NB, D), jnp.float32),
        scratch_types=[
            pltpu.VMEM((H, b_per_w), jnp.int32),
            pltpu.VMEM((NBUF, b_per_w, D), jnp.float32),
            pltpu.VMEM_SHARED((V, D), jnp.float32),
            pltpu.SemaphoreType.DMA((NBUF,)),
            pltpu.SemaphoreType.DMA((NBUF,)),
        ],
    )
    def gather_kernel(table_hbm, idx_hbm, out_hbm, idx_v, rows_v,
                      table_sp, gsem, wsem):
        wid = lax.axis_index("s") * NUM_CORES + lax.axis_index("c")
        b0 = pl.multiple_of(wid * b_per_w, b_per_w)

        @pl.when(lax.axis_index("s") == 0)
        def _stage_table():
            pltpu.sync_copy(table_hbm, table_sp)

        pltpu.sync_copy(idx_hbm.at[wid], idx_v)
        plsc.subcore_barrier()

        def start_gather(h, buf):
            pltpu.async_copy(table_sp.at[idx_v.at[h]], rows_v.at[buf],
                             gsem.at[buf])

        def wait_gather(buf):
            pltpu.make_async_copy(table_sp.at[idx_v.at[0]], rows_v.at[buf],
                                  gsem.at[buf]).wait()

        def start_write(h, buf):
            pltpu.async_copy(rows_v.at[buf],
                             out_hbm.at[h, pl.ds(b0, b_per_w)], wsem.at[buf])

        def wait_write(buf):
            pltpu.make_async_copy(rows_v.at[buf],
                                  out_hbm.at[0, pl.ds(b0, b_per_w)],
                                  wsem.at[buf]).wait()

        for i in range(NBUF - 1):
            start_gather(i, i)

        @pl.loop(0, H)
        def step(h):
            b = lax.rem(h, NBUF)
            bg = lax.rem(h + NBUF - 1, NBUF)
            in_range = h + NBUF - 1 < H

            @pl.when(jnp.logical_and(in_range, h >= 1))
            def _drain():
                wait_write(bg)

            @pl.when(in_range)
            def _prefetch():
                start_gather(h + NBUF - 1, bg)

            wait_gather(b)
            start_write(h, b)

        for i in range(NBUF):
            wait_write(i)

    return gather_kernel


def kernel(words, table):
    NB, H = words.shape
    V, D = table.shape
    b_per_w = NB // NUM_WORKERS
    idx = words.reshape(NUM_WORKERS, b_per_w, H).transpose(0, 2, 1)
    out = _make_gather(NB, H, D, V)(table, idx)
    return out.transpose(1, 0, 2)
